# Initial kernel scaffold; baseline (speedup 1.0000x reference)
import jax
import jax.numpy as jnp
from jax.experimental import pallas as pl


def kernel(x, Wr, br, W1, b1, W2, b2):
    raise NotImplementedError("write your pallas kernel here")


# speedup vs baseline: 2.2053x; 2.2053x over previous
import functools

import jax
import jax.numpy as jnp
from jax import lax
from jax.experimental import pallas as pl
from jax.experimental.pallas import tpu as pltpu
from jax.experimental.pallas import tpu_sc as plsc

D = 768
E = 64
T = 2048
TOPK = 2
TR = 128
MAX_TILES = T * TOPK // TR + E - 1
MAX_TILES = 96
T_PAD = MAX_TILES * TR
NW = 32


def _router_body(x_ref, wr_ref, br_ref, wts_ref, idx_ref):
    logits = jnp.dot(x_ref[...], wr_ref[...], preferred_element_type=jnp.float32)
    logits = logits + br_ref[...]
    iota = lax.broadcasted_iota(jnp.int32, (T, E), 1)
    m1 = jnp.max(logits, axis=-1, keepdims=True)
    i1 = jnp.min(jnp.where(logits == m1, iota, E), axis=-1, keepdims=True)
    masked = jnp.where(iota == i1, -jnp.inf, logits)
    m2 = jnp.max(masked, axis=-1, keepdims=True)
    i2 = jnp.min(jnp.where(masked == m2, iota, E), axis=-1, keepdims=True)
    e2 = jnp.exp(m2 - m1)
    s = 1.0 + e2
    wts_ref[...] = jnp.concatenate([1.0 / s, e2 / s], axis=-1)
    idx_ref[...] = jnp.concatenate([i1, i2], axis=-1)


def _router(x, Wr, br, interpret=False):
    return pl.pallas_call(
        _router_body,
        out_shape=[
            jax.ShapeDtypeStruct((T, TOPK), jnp.float32),
            jax.ShapeDtypeStruct((T, TOPK), jnp.int32),
        ],
        interpret=interpret,
    )(x, Wr, br.reshape(1, E))


def _plan(idx, wts):
    e_flat = idx.reshape(-1).astype(jnp.int32)
    perm = jnp.argsort(e_flat)
    e_sorted = e_flat[perm]
    counts = jnp.bincount(e_flat, length=E)
    tiles_e = (counts + TR - 1) // TR
    cum_tiles = jnp.cumsum(tiles_e)
    n_tiles = cum_tiles[-1:].astype(jnp.int32)
    off = jnp.cumsum(counts) - counts
    padded_off = (cum_tiles - tiles_e) * TR
    ranks = jnp.arange(2 * T, dtype=jnp.int32) - off[e_sorted].astype(jnp.int32)
    dest = padded_off[e_sorted].astype(jnp.int32) + ranks
    src_token = jnp.zeros((T_PAD,), jnp.int32).at[dest].set(
        (perm // TOPK).astype(jnp.int32))
    pos = jnp.zeros((2 * T,), jnp.int32).at[perm].set(dest)
    te_raw = jnp.searchsorted(cum_tiles, jnp.arange(MAX_TILES), side="right")
    last_e = e_sorted[-1]
    tile_expert = jnp.where(jnp.arange(MAX_TILES) < n_tiles[0], te_raw,
                            last_e).astype(jnp.int32)
    return (src_token, tile_expert, n_tiles,
            pos[0::2], pos[1::2], wts[:, 0], wts[:, 1])


_SC_MESH = dict(core_axis_name="c", subcore_axis_name="s")
_G_ROWS = T_PAD // NW
_G_CHUNK = 128
_G_NCHUNK = _G_ROWS // _G_CHUNK


def _gather_sc(x, src_token):
    @functools.partial(
        pl.kernel,
        mesh=plsc.VectorSubcoreMesh(**_SC_MESH),
        out_type=jax.ShapeDtypeStruct((T_PAD, D), jnp.float32),
        scratch_types=[
            pltpu.VMEM((_G_CHUNK,), jnp.int32),
            pltpu.VMEM((_G_CHUNK, D), jnp.float32),
            pltpu.SemaphoreType.DMA,
        ],
    )
    def k(x_hbm, src_hbm, out_hbm, idx_v, rows_v, sem):
        wid = lax.axis_index("s") * 2 + lax.axis_index("c")
        for c in range(_G_NCHUNK):
            base = wid * _G_ROWS + c * _G_CHUNK
            pltpu.sync_copy(src_hbm.at[pl.ds(base, _G_CHUNK)], idx_v)
            pltpu.async_copy(x_hbm.at[idx_v], rows_v, sem).wait()
            pltpu.sync_copy(rows_v, out_hbm.at[pl.ds(base, _G_CHUNK)])

    return k(x, src_token)


def _mlp_body(te_ref, nt_ref, xs_ref, w1_ref, b1_ref, w2_ref, b2_ref, ys_ref):
    j = pl.program_id(0)

    @pl.when(j < nt_ref[0])
    def _():
        xt = xs_ref[...]
        h = jnp.dot(xt, w1_ref[0], preferred_element_type=jnp.float32)
        h = jnp.maximum(h + b1_ref[0, 0, :], 0.0)
        y = jnp.dot(h, w2_ref[0], preferred_element_type=jnp.float32)
        ys_ref[...] = y + b2_ref[0, 0, :]


def _mlp(tile_expert, n_tiles, xs, W1, b1, W2, b2, interpret=False):
    grid_spec = pltpu.PrefetchScalarGridSpec(
        num_scalar_prefetch=2,
        grid=(MAX_TILES,),
        in_specs=[
            pl.BlockSpec((TR, D), lambda j, te, nt: (j, 0)),
            pl.BlockSpec((1, D, 4 * D), lambda j, te, nt: (te[j], 0, 0)),
            pl.BlockSpec((1, 1, 4 * D), lambda j, te, nt: (te[j], 0, 0)),
            pl.BlockSpec((1, 4 * D, D), lambda j, te, nt: (te[j], 0, 0)),
            pl.BlockSpec((1, 1, D), lambda j, te, nt: (te[j], 0, 0)),
        ],
        out_specs=pl.BlockSpec((TR, D), lambda j, te, nt: (j, 0)),
    )
    return pl.pallas_call(
        _mlp_body,
        grid_spec=grid_spec,
        out_shape=jax.ShapeDtypeStruct((T_PAD, D), jnp.float32),
        compiler_params=pltpu.CompilerParams(
            dimension_semantics=("arbitrary",),
            vmem_limit_bytes=120 * 1024 * 1024,
        ),
        interpret=interpret,
    )(tile_expert, n_tiles, xs, W1, b1.reshape(E, 1, 4 * D), W2,
      b2.reshape(E, 1, D))


_C_ROWS = T // NW
_LANES = 16


def _combine_sc(ys, posA, posB, wA, wB):
    @functools.partial(
        pl.kernel,
        mesh=plsc.VectorSubcoreMesh(**_SC_MESH),
        out_type=jax.ShapeDtypeStruct((T, D), jnp.float32),
        compiler_params=pltpu.CompilerParams(needs_layout_passes=False),
        scratch_types=[
            pltpu.VMEM((_C_ROWS,), jnp.int32),
            pltpu.VMEM((_C_ROWS,), jnp.int32),
            pltpu.VMEM((_C_ROWS,), jnp.float32),
            pltpu.VMEM((_C_ROWS,), jnp.float32),
            pltpu.VMEM((_C_ROWS, D), jnp.float32),
            pltpu.VMEM((_C_ROWS, D), jnp.float32),
            pltpu.SemaphoreType.DMA,
        ],
    )
    def k(ys_hbm, pa_hbm, pb_hbm, wa_hbm, wb_hbm, out_hbm,
          ia, ib, va, vb, ra, rb, sem):
        wid = lax.axis_index("s") * 2 + lax.axis_index("c")
        base = wid * _C_ROWS
        pltpu.sync_copy(pa_hbm.at[pl.ds(base, _C_ROWS)], ia)
        pltpu.sync_copy(pb_hbm.at[pl.ds(base, _C_ROWS)], ib)
        pltpu.sync_copy(wa_hbm.at[pl.ds(base, _C_ROWS)], va)
        pltpu.sync_copy(wb_hbm.at[pl.ds(base, _C_ROWS)], vb)
        ca = pltpu.async_copy(ys_hbm.at[ia], ra, sem)
        cb = pltpu.async_copy(ys_hbm.at[ib], rb, sem)
        ca.wait()
        cb.wait()

        def body(r, carry):
            ridx = jnp.full((_LANES,), r, jnp.int32)
            a = plsc.load_gather(va, [ridx])
            b = plsc.load_gather(vb, [ridx])
            for j in range(D // _LANES):
                s = pl.ds(j * _LANES, _LANES)
                ra[r, s] = a * ra[r, s] + b * rb[r, s]
            return carry

        lax.fori_loop(0, _C_ROWS, body, 0)
        pltpu.sync_copy(ra, out_hbm.at[pl.ds(base, _C_ROWS)])

    return k(ys, posA, posB, wA, wB)


def kernel(x, Wr, br, W1, b1, W2, b2):
    wts, idx = _router(x, Wr, br)
    src_token, tile_expert, n_tiles, posA, posB, wA, wB = _plan(idx, wts)
    xs = _gather_sc(x, src_token)
    ys = _mlp(tile_expert, n_tiles, xs, W1, b1, W2, b2)
    return _combine_sc(ys, posA, posB, wA, wB)


# speedup vs baseline: 2.4603x; 1.1156x over previous
import functools

import jax
import jax.numpy as jnp
from jax import lax
from jax.experimental import pallas as pl
from jax.experimental.pallas import tpu as pltpu
from jax.experimental.pallas import tpu_sc as plsc

D = 768
E = 64
T = 2048
TOPK = 2
TR = 64
MAX_TILES = T * TOPK // TR + E
T_PAD = MAX_TILES * TR
NW = 32


def _router_body(x_ref, wr_ref, br_ref, wts_ref, idx_ref):
    logits = jnp.dot(x_ref[...], wr_ref[...], preferred_element_type=jnp.float32)
    logits = logits + br_ref[...]
    iota = lax.broadcasted_iota(jnp.int32, (T, E), 1)
    m1 = jnp.max(logits, axis=-1, keepdims=True)
    i1 = jnp.min(jnp.where(logits == m1, iota, E), axis=-1, keepdims=True)
    masked = jnp.where(iota == i1, -jnp.inf, logits)
    m2 = jnp.max(masked, axis=-1, keepdims=True)
    i2 = jnp.min(jnp.where(masked == m2, iota, E), axis=-1, keepdims=True)
    e2 = jnp.exp(m2 - m1)
    s = 1.0 + e2
    wts_ref[...] = jnp.concatenate([1.0 / s, e2 / s], axis=-1)
    idx_ref[...] = jnp.concatenate([i1, i2], axis=-1)


def _router(x, Wr, br, interpret=False):
    return pl.pallas_call(
        _router_body,
        out_shape=[
            jax.ShapeDtypeStruct((T, TOPK), jnp.float32),
            jax.ShapeDtypeStruct((T, TOPK), jnp.int32),
        ],
        interpret=interpret,
    )(x, Wr, br.reshape(1, E))


def _plan(idx, wts):
    e_flat = idx.reshape(-1).astype(jnp.int32)
    perm = jnp.argsort(e_flat)
    e_sorted = e_flat[perm]
    counts = jnp.bincount(e_flat, length=E)
    tiles_e = (counts + TR - 1) // TR
    cum_tiles = jnp.cumsum(tiles_e)
    n_tiles = cum_tiles[-1:].astype(jnp.int32)
    off = jnp.cumsum(counts) - counts
    padded_off = (cum_tiles - tiles_e) * TR
    ranks = jnp.arange(2 * T, dtype=jnp.int32) - off[e_sorted].astype(jnp.int32)
    dest = padded_off[e_sorted].astype(jnp.int32) + ranks
    src_token = jnp.zeros((T_PAD,), jnp.int32).at[dest].set(
        (perm // TOPK).astype(jnp.int32))
    pos = jnp.zeros((2 * T,), jnp.int32).at[perm].set(dest)
    te_raw = jnp.searchsorted(cum_tiles, jnp.arange(MAX_TILES), side="right")
    last_e = e_sorted[-1]
    tile_expert = jnp.where(jnp.arange(MAX_TILES) < n_tiles[0], te_raw,
                            last_e).astype(jnp.int32)
    return (src_token, tile_expert, n_tiles,
            pos[0::2], pos[1::2], wts[:, 0], wts[:, 1])


_SC_MESH = dict(core_axis_name="c", subcore_axis_name="s")
_G_ROWS = T_PAD // NW
_G_CHUNK = 32
_G_NBUF = 4
_G_NCHUNK = _G_ROWS // _G_CHUNK


def _gather_sc(x, src_token):
    @functools.partial(
        pl.kernel,
        mesh=plsc.VectorSubcoreMesh(**_SC_MESH),
        out_type=jax.ShapeDtypeStruct((T_PAD, D), jnp.float32),
        scratch_types=(
            [pltpu.VMEM((_G_ROWS,), jnp.int32)]
            + [pltpu.VMEM((_G_CHUNK, D), jnp.float32)] * _G_NBUF
            + [pltpu.SemaphoreType.DMA] * _G_NBUF
        ),
    )
    def k(x_hbm, src_hbm, out_hbm, idx_v, *bufs_sems):
        bufs = bufs_sems[:_G_NBUF]
        sems = bufs_sems[_G_NBUF:]
        wid = lax.axis_index("s") * 2 + lax.axis_index("c")
        base = wid * _G_ROWS
        pltpu.sync_copy(src_hbm.at[pl.ds(base, _G_ROWS)], idx_v)
        cps = [None] * _G_NBUF
        for c in range(_G_NBUF):
            cps[c] = pltpu.async_copy(
                x_hbm.at[idx_v.at[pl.ds(c * _G_CHUNK, _G_CHUNK)]],
                bufs[c], sems[c])
        for c in range(_G_NCHUNK):
            b = c % _G_NBUF
            cps[b].wait()
            pltpu.sync_copy(bufs[b], out_hbm.at[pl.ds(base + c * _G_CHUNK,
                                                      _G_CHUNK)])
            nc = c + _G_NBUF
            if nc < _G_NCHUNK:
                cps[b] = pltpu.async_copy(
                    x_hbm.at[idx_v.at[pl.ds(nc * _G_CHUNK, _G_CHUNK)]],
                    bufs[b], sems[b])

    return k(x, src_token)


def _mlp_body(te_ref, nt_ref, xs_ref, w1_ref, b1_ref, w2_ref, b2_ref, ys_ref):
    j = pl.program_id(0)

    @pl.when(j < nt_ref[0])
    def _():
        xt = xs_ref[...]
        h = jnp.dot(xt, w1_ref[0], preferred_element_type=jnp.float32)
        h = jnp.maximum(h + b1_ref[0, 0, :], 0.0)
        y = jnp.dot(h, w2_ref[0], preferred_element_type=jnp.float32)
        ys_ref[...] = y + b2_ref[0, 0, :]


def _mlp(tile_expert, n_tiles, xs, W1, b1, W2, b2, interpret=False):
    grid_spec = pltpu.PrefetchScalarGridSpec(
        num_scalar_prefetch=2,
        grid=(MAX_TILES,),
        in_specs=[
            pl.BlockSpec((TR, D), lambda j, te, nt: (j, 0)),
            pl.BlockSpec((1, D, 4 * D), lambda j, te, nt: (te[j], 0, 0)),
            pl.BlockSpec((1, 1, 4 * D), lambda j, te, nt: (te[j], 0, 0)),
            pl.BlockSpec((1, 4 * D, D), lambda j, te, nt: (te[j], 0, 0)),
            pl.BlockSpec((1, 1, D), lambda j, te, nt: (te[j], 0, 0)),
        ],
        out_specs=pl.BlockSpec((TR, D), lambda j, te, nt: (j, 0)),
    )
    return pl.pallas_call(
        _mlp_body,
        grid_spec=grid_spec,
        out_shape=jax.ShapeDtypeStruct((T_PAD, D), jnp.float32),
        compiler_params=pltpu.CompilerParams(
            dimension_semantics=("arbitrary",),
            vmem_limit_bytes=120 * 1024 * 1024,
        ),
        interpret=interpret,
    )(tile_expert, n_tiles, xs, W1, b1.reshape(E, 1, 4 * D), W2,
      b2.reshape(E, 1, D))


_C_ROWS = T // NW
_LANES = 16


def _combine_sc(ys, posA, posB, wA, wB):
    @functools.partial(
        pl.kernel,
        mesh=plsc.VectorSubcoreMesh(**_SC_MESH),
        out_type=jax.ShapeDtypeStruct((T, D), jnp.float32),
        compiler_params=pltpu.CompilerParams(needs_layout_passes=False),
        scratch_types=[
            pltpu.VMEM((_C_ROWS,), jnp.int32),
            pltpu.VMEM((_C_ROWS,), jnp.int32),
            pltpu.VMEM((_C_ROWS,), jnp.float32),
            pltpu.VMEM((_C_ROWS,), jnp.float32),
            pltpu.VMEM((_C_ROWS, D), jnp.float32),
            pltpu.VMEM((_C_ROWS, D), jnp.float32),
            pltpu.SemaphoreType.DMA,
        ],
    )
    def k(ys_hbm, pa_hbm, pb_hbm, wa_hbm, wb_hbm, out_hbm,
          ia, ib, va, vb, ra, rb, sem):
        wid = lax.axis_index("s") * 2 + lax.axis_index("c")
        base = wid * _C_ROWS
        pltpu.sync_copy(pa_hbm.at[pl.ds(base, _C_ROWS)], ia)
        pltpu.sync_copy(pb_hbm.at[pl.ds(base, _C_ROWS)], ib)
        pltpu.sync_copy(wa_hbm.at[pl.ds(base, _C_ROWS)], va)
        pltpu.sync_copy(wb_hbm.at[pl.ds(base, _C_ROWS)], vb)
        ca = pltpu.async_copy(ys_hbm.at[ia], ra, sem)
        cb = pltpu.async_copy(ys_hbm.at[ib], rb, sem)
        ca.wait()
        cb.wait()

        def body(r, carry):
            ridx = jnp.full((_LANES,), r, jnp.int32)
            a = plsc.load_gather(va, [ridx])
            b = plsc.load_gather(vb, [ridx])
            for j in range(D // _LANES):
                s = pl.ds(j * _LANES, _LANES)
                ra[r, s] = a * ra[r, s] + b * rb[r, s]
            return carry

        lax.fori_loop(0, _C_ROWS, body, 0)
        pltpu.sync_copy(ra, out_hbm.at[pl.ds(base, _C_ROWS)])

    return k(ys, posA, posB, wA, wB)


def kernel(x, Wr, br, W1, b1, W2, b2):
    wts, idx = _router(x, Wr, br)
    src_token, tile_expert, n_tiles, posA, posB, wA, wB = _plan(idx, wts)
    xs = _gather_sc(x, src_token)
    ys = _mlp(tile_expert, n_tiles, xs, W1, b1, W2, b2)
    return _combine_sc(ys, posA, posB, wA, wB)
